# initial kernel scaffold (unmeasured)
import numpy as np

import jax
import jax.numpy as jnp
from jax import lax
from jax.experimental import pallas as pl
from jax.experimental.pallas import tpu as pltpu

N_DEV = 4
B, SQ, SKV_SH, DH = 2, 512, 512, 64
H_SH = 8
HID = H_SH * DH
SKV = N_DEV * SKV_SH
D_OUT = 768
BLK = 64

BF = jnp.bfloat16
F32 = jnp.float32


def _global_mask() -> np.ndarray:
    qb = (np.arange(SQ) // BLK)[:, None]
    kb = (np.arange(SKV) // BLK)[None, :]
    m = (qb == kb) | (kb == 0) | ((qb + kb) % 3 == 0)
    return m.astype(np.float32)


def kernel(x, Wq, K_ext, V_ext, Wo):
    K2 = K_ext.reshape(B, SKV_SH, N_DEV * HID)
    V2 = V_ext.reshape(B, SKV_SH, N_DEV * HID)
    mask = jnp.asarray(_global_mask())

    def body(x_ref, wq_ref, k_ref, v_ref, wo_ref, mask_ref, out_ref,
             kvsend, kvrecv, orecv,
             kv_send_sems, kv_recv_sems, o_send_sems, o_recv_sems):
        my = lax.axis_index("i")

        for d in range(1, N_DEV):
            dst = lax.rem(my + d, N_DEV)
            kvsend[d - 1, 0] = k_ref[:, :, pl.ds(dst * HID, HID)].astype(BF)
            kvsend[d - 1, 1] = v_ref[:, :, pl.ds(dst * HID, HID)].astype(BF)
        kv_rdmas = []
        for d in range(1, N_DEV):
            dst = lax.rem(my + d, N_DEV)
            r = pltpu.make_async_remote_copy(
                src_ref=kvsend.at[d - 1],
                dst_ref=kvrecv.at[d - 1],
                send_sem=kv_send_sems.at[d - 1],
                recv_sem=kv_recv_sems.at[d - 1],
                device_id=(dst,),
                device_id_type=pl.DeviceIdType.MESH,
            )
            r.start()
            kv_rdmas.append(r)

        wq_bf = wq_ref[:, :].astype(BF)
        wo_bf = wo_ref[:, :].astype(BF)

        for r in kv_rdmas:
            r.wait()

        for b in range(B):
            xb = x_ref[b].astype(BF)
            qb = lax.dot_general(xb, wq_bf, (((1,), (0,)), ((), ())),
                                 preferred_element_type=F32).astype(BF)
            kloc = k_ref[b, :, pl.ds(my * HID, HID)].astype(BF)
            vloc = v_ref[b, :, pl.ds(my * HID, HID)].astype(BF)
            ctxs = []
            for h in range(H_SH):
                q = qb[:, h * DH:(h + 1) * DH]
                parts = []
                for d in range(N_DEV):
                    if d == 0:
                        src = my
                        kc = kloc[:, h * DH:(h + 1) * DH]
                        vc = vloc[:, h * DH:(h + 1) * DH]
                    else:
                        src = lax.rem(my - d + N_DEV, N_DEV)
                        kc = kvrecv[d - 1, 0, b, :, h * DH:(h + 1) * DH]
                        vc = kvrecv[d - 1, 1, b, :, h * DH:(h + 1) * DH]
                    s = lax.dot_general(q, kc, (((1,), (1,)), ((), ())),
                                        preferred_element_type=F32) * 0.125
                    mk = mask_ref[:, pl.ds(src * SKV_SH, SKV_SH)]
                    s = jnp.where(mk > 0.5, s, -1e9)
                    parts.append((s, vc))
                m = parts[0][0].max(axis=1, keepdims=True)
                for s, _ in parts[1:]:
                    m = jnp.maximum(m, s.max(axis=1, keepdims=True))
                l = None
                ctx = None
                for s, vc in parts:
                    p = jnp.exp(s - m)
                    ls = p.sum(axis=1, keepdims=True)
                    c = lax.dot_general(p.astype(BF), vc,
                                        (((1,), (0,)), ((), ())),
                                        preferred_element_type=F32)
                    l = ls if l is None else l + ls
                    ctx = c if ctx is None else ctx + c
                ctxs.append(ctx / l)
            ctx_b = jnp.concatenate(ctxs, axis=1).astype(BF)
            out_ref[b] = lax.dot_general(ctx_b, wo_bf,
                                         (((1,), (0,)), ((), ())),
                                         preferred_element_type=F32)

        o_rdmas = []
        for d in range(1, N_DEV):
            dst = lax.rem(my + d, N_DEV)
            r = pltpu.make_async_remote_copy(
                src_ref=out_ref,
                dst_ref=orecv.at[d - 1],
                send_sem=o_send_sems.at[d - 1],
                recv_sem=o_recv_sems.at[d - 1],
                device_id=(dst,),
                device_id_type=pl.DeviceIdType.MESH,
            )
            r.start()
            o_rdmas.append(r)
        for r in o_rdmas:
            r.wait()
        out_ref[:, :, :] = (out_ref[:, :, :] + orecv[0] + orecv[1]
                            + orecv[2])

    return pl.pallas_call(
        body,
        out_shape=jax.ShapeDtypeStruct((B, SQ, D_OUT), F32),
        in_specs=[pl.BlockSpec(memory_space=pltpu.VMEM)] * 6,
        out_specs=pl.BlockSpec(memory_space=pltpu.VMEM),
        scratch_shapes=[
            pltpu.VMEM((N_DEV - 1, 2, B, SKV_SH, HID), BF),
            pltpu.VMEM((N_DEV - 1, 2, B, SKV_SH, HID), BF),
            pltpu.VMEM((N_DEV - 1, B, SQ, D_OUT), F32),
            pltpu.SemaphoreType.DMA((N_DEV - 1,)),
            pltpu.SemaphoreType.DMA((N_DEV - 1,)),
            pltpu.SemaphoreType.DMA((N_DEV - 1,)),
            pltpu.SemaphoreType.DMA((N_DEV - 1,)),
        ],
    )(x, Wq, K2, V2, Wo, mask)


# baseline (device time: 160432 ns/iter reference)
import numpy as np

import jax
import jax.numpy as jnp
from jax import lax
from jax.experimental import pallas as pl
from jax.experimental.pallas import tpu as pltpu

N_DEV = 4
B, SQ, SKV_SH, DH = 2, 512, 512, 64
H_SH = 8
HID = H_SH * DH
SKV = N_DEV * SKV_SH
D_OUT = 768
BLK = 64

BF = jnp.bfloat16
F32 = jnp.float32


def _global_mask() -> np.ndarray:
    qb = (np.arange(SQ) // BLK)[:, None]
    kb = (np.arange(SKV) // BLK)[None, :]
    m = (qb == kb) | (kb == 0) | ((qb + kb) % 3 == 0)
    return m.astype(np.float32)


def kernel(x, Wq, K_ext, V_ext, Wo):
    K2 = K_ext.reshape(B, SKV_SH, N_DEV * HID)
    V2 = V_ext.reshape(B, SKV_SH, N_DEV * HID)
    mask = jnp.asarray(_global_mask(), dtype=BF)

    def body(x_ref, wq_ref, k_ref, v_ref, wo_ref, mask_ref, out_ref,
             kvsend, kvall, qbuf, ctxbuf, osend, orecv,
             kv_send_sems, kv_recv_sems, o_send_sems, o_recv_sems):
        my = lax.axis_index("i")

        for d in range(1, N_DEV):
            dst = lax.rem(my + d, N_DEV)
            kvsend[d - 1, 0] = k_ref[:, :, pl.ds(dst * HID, HID)].astype(BF)
            kvsend[d - 1, 1] = v_ref[:, :, pl.ds(dst * HID, HID)].astype(BF)
        kv_rdmas = []
        for d in range(1, N_DEV):
            dst = lax.rem(my + d, N_DEV)
            r = pltpu.make_async_remote_copy(
                src_ref=kvsend.at[d - 1],
                dst_ref=kvall.at[d],
                send_sem=kv_send_sems.at[d - 1],
                recv_sem=kv_recv_sems.at[d - 1],
                device_id=(dst,),
                device_id_type=pl.DeviceIdType.MESH,
            )
            r.start()
            kv_rdmas.append(r)

        kvall[0, 0] = k_ref[:, :, pl.ds(my * HID, HID)].astype(BF)
        kvall[0, 1] = v_ref[:, :, pl.ds(my * HID, HID)].astype(BF)
        wq_bf = wq_ref[:, :].astype(BF)
        wo_bf = wo_ref[:, :].astype(BF)
        for b in range(B):
            xb = x_ref[b].astype(BF)
            qbuf[b] = lax.dot_general(
                xb, wq_bf, (((1,), (0,)), ((), ())),
                preferred_element_type=F32).astype(BF)

        for r in kv_rdmas:
            r.wait()

        for b in range(B):
            def pair_step(hp, carry, b=b):
                hs = hp * (2 * DH)
                q2 = qbuf[b, :, pl.ds(hs, 2 * DH)]
                chunks = []
                for d in range(N_DEV):
                    src = lax.rem(my - d + N_DEV, N_DEV)
                    kc2 = kvall[d, 0, b, :, pl.ds(hs, 2 * DH)]
                    vc2 = kvall[d, 1, b, :, pl.ds(hs, 2 * DH)]
                    mk = mask_ref[:, pl.ds(src * SKV_SH, SKV_SH)]
                    chunks.append((kc2, vc2, mk))
                outs = []
                for sub in range(2):
                    lo, hi = sub * DH, (sub + 1) * DH
                    q = q2[:, lo:hi]
                    m = jnp.full((SQ, 1), -1e30, F32)
                    l = jnp.zeros((SQ, 1), F32)
                    ctx = jnp.zeros((SQ, DH), F32)
                    for kc2, vc2, mk in chunks:
                        kc = kc2[:, lo:hi]
                        vc = vc2[:, lo:hi]
                        s = lax.dot_general(q, kc,
                                            (((1,), (1,)), ((), ())),
                                            preferred_element_type=F32) * 0.125
                        s = jnp.where(mk > 0.5, s, -1e9)
                        m_new = jnp.maximum(m, s.max(axis=1, keepdims=True))
                        alpha = jnp.exp(m - m_new)
                        p = jnp.exp(s - m_new)
                        l = l * alpha + p.sum(axis=1, keepdims=True)
                        ctx = ctx * alpha + lax.dot_general(
                            p.astype(BF), vc, (((1,), (0,)), ((), ())),
                            preferred_element_type=F32)
                        m = m_new
                    outs.append(ctx / l)
                ctxbuf[b, :, pl.ds(hs, 2 * DH)] = jnp.concatenate(
                    outs, axis=1).astype(BF)
                return carry

            lax.fori_loop(0, H_SH // 2, pair_step, 0)

        for b in range(B):
            out_ref[b] = lax.dot_general(ctxbuf[b], wo_bf,
                                         (((1,), (0,)), ((), ())),
                                         preferred_element_type=F32)
            osend[b] = out_ref[b].astype(BF)

        o_rdmas = []
        for d in range(1, N_DEV):
            dst = lax.rem(my + d, N_DEV)
            r = pltpu.make_async_remote_copy(
                src_ref=osend,
                dst_ref=orecv.at[d - 1],
                send_sem=o_send_sems.at[d - 1],
                recv_sem=o_recv_sems.at[d - 1],
                device_id=(dst,),
                device_id_type=pl.DeviceIdType.MESH,
            )
            r.start()
            o_rdmas.append(r)
        for r in o_rdmas:
            r.wait()
        out_ref[:, :, :] = (out_ref[:, :, :]
                            + orecv[0].astype(F32)
                            + orecv[1].astype(F32)
                            + orecv[2].astype(F32))

    return pl.pallas_call(
        body,
        out_shape=jax.ShapeDtypeStruct((B, SQ, D_OUT), F32),
        in_specs=[pl.BlockSpec(memory_space=pltpu.VMEM)] * 6,
        out_specs=pl.BlockSpec(memory_space=pltpu.VMEM),
        scratch_shapes=[
            pltpu.VMEM((N_DEV - 1, 2, B, SKV_SH, HID), BF),
            pltpu.VMEM((N_DEV, 2, B, SKV_SH, HID), BF),
            pltpu.VMEM((B, SQ, HID), BF),
            pltpu.VMEM((B, SQ, HID), BF),
            pltpu.VMEM((B, SQ, D_OUT), BF),
            pltpu.VMEM((N_DEV - 1, B, SQ, D_OUT), BF),
            pltpu.SemaphoreType.DMA((N_DEV - 1,)),
            pltpu.SemaphoreType.DMA((N_DEV - 1,)),
            pltpu.SemaphoreType.DMA((N_DEV - 1,)),
            pltpu.SemaphoreType.DMA((N_DEV - 1,)),
        ],
        compiler_params=pltpu.CompilerParams(
            vmem_limit_bytes=100 * 1024 * 1024,
        ),
    )(x, Wq, K2, V2, Wo, mask)
